# Initial kernel scaffold; baseline (speedup 1.0000x reference)
import jax
import jax.numpy as jnp
from jax.experimental import pallas as pl


def kernel(invariant_node, equivariant_node_F, equivariant_node_f, equivariant_node_dr, invariant_edge, neighbor_mask, distances, distance_vectors, neighbor_indices, params):
    raise NotImplementedError("write your pallas kernel here")


# speedup vs baseline: 5.0607x; 5.0607x over previous
import jax
import jax.numpy as jnp
from jax.experimental import pallas as pl
from jax.experimental.pallas import tpu as pltpu

_CUTOFF = 5.0


def _dot(a, b):
    return jnp.dot(a, b, preferred_element_type=jnp.float32)


def _fused(T, K, F,
           x_full_ref, x_tile_ref, eqF_ref, eqf_ref, drtab_ref, drtile_ref,
           edge_ref, mask_ref, d_ref, dvec_ref, idx_ref,
           WimeT, bime, W1i, b1i, W2i, b2i, Wemc,
           W1emf, b1emf, W2emf, b2emf,
           W1eme, W2eme,
           W1esc, b1esc, W2esc, b2esc,
           W1isc, b1isc, W2isc, b2isc,
           out_x, out_eqF, out_eqf, out_eqdr):
    N = x_full_ref.shape[1]
    TK = T * K

    def mlp2(x, W1, b1, W2, b2):
        h = _dot(x, W1[...]) + b1[...]
        h = h * jax.nn.sigmoid(h)
        return _dot(h, W2[...]) + b2[...]

    x_full = x_full_ref[0]
    x_tile = x_tile_ref[0]
    imn_full = mlp2(x_full, W1i, b1i, W2i, b2i)
    imn_tile = mlp2(x_tile, W1i, b1i, W2i, b2i)

    ime = _dot(edge_ref[0], WimeT[...]) + bime[...]
    d = d_ref[0]
    cut = 0.5 * (jnp.cos(jnp.pi * d / _CUTOFF) + 1.0)
    cut = cut * (d < _CUTOFF).astype(jnp.float32)
    ime = ime * cut

    idx = idx_ref[0]
    iota = jax.lax.broadcasted_iota(jnp.int32, (TK, N), 1)
    onehot = (idx == iota).astype(jnp.float32)
    imn_f = _dot(onehot, imn_full)
    dr_f = _dot(onehot, drtab_ref[0])

    imn_i = jnp.broadcast_to(imn_tile[:, None, :], (T, K, F)).reshape(TK, F)
    msg = ime * imn_i * imn_f
    mask = mask_ref[0]
    x_new = x_tile + (msg * mask).reshape(T, K, F).sum(axis=1)

    s = _dot(msg, Wemc[...])
    w3 = (s * mask) * dvec_ref[0]
    eqF_new = eqF_ref[0] + w3.reshape(T, K, 3).sum(axis=1)

    emf = mlp2(msg, W1emf, b1emf, W2emf, b2emf)
    h = _dot(msg, W1eme[...])
    h = h * jax.nn.sigmoid(h)
    eme = _dot(h, W2eme[...]) * mask

    esc = mlp2(x_new, W1esc, b1esc, W2esc, b2esc)
    isc = mlp2(x_new, W1isc, b1isc, W2isc, b2isc)

    dot3 = jnp.zeros((T, F), jnp.float32)
    for c in range(3):
        w_c = w3[:, c:c + 1]
        eqf_c = eqf_ref[0, :, c, :] + (emf * w_c).reshape(T, K, F).sum(axis=1)
        dr_c = drtile_ref[0][:, c * F:(c + 1) * F]
        gdr_c = dr_f[:, c * F:(c + 1) * F]
        eqdr_c = dr_c + (eme * gdr_c).reshape(T, K, F).sum(axis=1)
        eqdr_c = eqdr_c + esc * eqf_c
        out_eqf[0, :, c, :] = eqf_c
        out_eqdr[0, :, c, :] = eqdr_c
        dot3 = dot3 + eqdr_c * eqf_c

    out_x[0] = x_new + isc * dot3
    out_eqF[0] = eqF_new


def kernel(invariant_node, equivariant_node_F, equivariant_node_f,
           equivariant_node_dr, invariant_edge, neighbor_mask, distances,
           distance_vectors, neighbor_indices, params):
    B, N, F = invariant_node.shape
    K = neighbor_indices.shape[-1]
    NB = invariant_edge.shape[-1]
    T = 64
    TK = T * K
    p = params

    edge_flat = invariant_edge.reshape(B, N * K, NB)
    d_col = distances.reshape(B, N * K, 1)
    mask_col = neighbor_mask.reshape(B, N * K, 1)
    idx_col = neighbor_indices.astype(jnp.int32).reshape(B, N * K, 1)
    dvec3 = distance_vectors.reshape(B, N * K, 3)
    dr_tab = equivariant_node_dr.reshape(B, N, 3 * F)

    def b2(v):
        return v.reshape(1, F)

    weights = [
        p["W_ime"].T, p["b_ime"].reshape(1, F),
        p["imn_W1"].T, b2(p["imn_b1"]), p["imn_W2"].T, b2(p["imn_b2"]),
        p["W_emc"].T,
        p["emf_W1"].T, b2(p["emf_b1"]), p["emf_W2"].T, b2(p["emf_b2"]),
        p["eme_W1"].T, p["eme_W2"].T,
        p["esc_W1"].T, b2(p["esc_b1"]), p["esc_W2"].T, b2(p["esc_b2"]),
        p["isc_W1"].T, b2(p["isc_b1"]), p["isc_W2"].T, b2(p["isc_b2"]),
    ]

    def wspec(w):
        shp = w.shape
        return pl.BlockSpec(shp, lambda b, t: (0,) * len(shp))

    grid = (B, N // T)
    in_specs = [
        pl.BlockSpec((1, N, F), lambda b, t: (b, 0, 0)),
        pl.BlockSpec((1, T, F), lambda b, t: (b, t, 0)),
        pl.BlockSpec((1, T, 3), lambda b, t: (b, t, 0)),
        pl.BlockSpec((1, T, 3, F), lambda b, t: (b, t, 0, 0)),
        pl.BlockSpec((1, N, 3 * F), lambda b, t: (b, 0, 0)),
        pl.BlockSpec((1, T, 3 * F), lambda b, t: (b, t, 0)),
        pl.BlockSpec((1, TK, NB), lambda b, t: (b, t, 0)),
        pl.BlockSpec((1, TK, 1), lambda b, t: (b, t, 0)),
        pl.BlockSpec((1, TK, 1), lambda b, t: (b, t, 0)),
        pl.BlockSpec((1, TK, 3), lambda b, t: (b, t, 0)),
        pl.BlockSpec((1, TK, 1), lambda b, t: (b, t, 0)),
    ] + [wspec(w) for w in weights]

    out_specs = [
        pl.BlockSpec((1, T, F), lambda b, t: (b, t, 0)),
        pl.BlockSpec((1, T, 3), lambda b, t: (b, t, 0)),
        pl.BlockSpec((1, T, 3, F), lambda b, t: (b, t, 0, 0)),
        pl.BlockSpec((1, T, 3, F), lambda b, t: (b, t, 0, 0)),
    ]
    out_shapes = [
        jax.ShapeDtypeStruct((B, N, F), jnp.float32),
        jax.ShapeDtypeStruct((B, N, 3), jnp.float32),
        jax.ShapeDtypeStruct((B, N, 3, F), jnp.float32),
        jax.ShapeDtypeStruct((B, N, 3, F), jnp.float32),
    ]

    import functools
    fn = functools.partial(_fused, T, K, F)
    outs = pl.pallas_call(
        fn,
        grid=grid,
        in_specs=in_specs,
        out_specs=out_specs,
        out_shape=out_shapes,
        compiler_params=pltpu.CompilerParams(
            dimension_semantics=("parallel", "parallel")),
    )(invariant_node, invariant_node, equivariant_node_F, equivariant_node_f,
      dr_tab, dr_tab, edge_flat, mask_col, d_col, dvec3, idx_col, *weights)

    return outs[0], outs[1], outs[2], outs[3]


# speedup vs baseline: 8.3649x; 1.6529x over previous
import jax
import jax.numpy as jnp
from jax.experimental import pallas as pl
from jax.experimental.pallas import tpu as pltpu

_CUTOFF = 5.0


def _dot(a, b):
    return jnp.dot(a, b, preferred_element_type=jnp.float32)


def _dotbf(a, b):
    return jnp.dot(a.astype(jnp.bfloat16), b.astype(jnp.bfloat16),
                   preferred_element_type=jnp.float32)


def _fused(T, K, F,
           x_full_ref, x_tile_ref, eqF_ref, eqf_ref, drtab_ref, drtile_ref,
           edge_ref, cols_ref,
           WimeT, bime, W1i, b1i, W2i, b2i, Wemcb,
           W1emf, b1emf, W2emf, b2emf,
           W1eme, W2eme,
           W1esc, b1esc, W2esc, b2esc,
           W1isc, b1isc, W2isc, b2isc,
           out_x, out_eqF, out_eqf, out_eqdr):
    N = x_full_ref.shape[1]
    TK = T * K

    def mlp2(x, W1, b1, W2, b2):
        h = _dot(x, W1[...]) + b1[...]
        h = h * jax.nn.sigmoid(h)
        return _dot(h, W2[...]) + b2[...]

    x_full = x_full_ref[0]
    x_tile = x_tile_ref[0]
    imn_full = mlp2(x_full, W1i, b1i, W2i, b2i)
    imn_tile = mlp2(x_tile, W1i, b1i, W2i, b2i)

    cols = cols_ref[0]
    pcol = jax.lax.broadcasted_iota(jnp.int32, (6, 5 * F + 2 * N), 1)
    prow = jax.lax.broadcasted_iota(jnp.int32, (6, 5 * F + 2 * N), 0)
    pmat = jnp.where(jnp.minimum(pcol // F, 5) == prow, 1.0, 0.0)
    bcast = _dotbf(cols, pmat)
    d_b = bcast[:, 0:F]
    mask_b = bcast[:, F:2 * F]
    dv_b = bcast[:, 2 * F:5 * F]
    idx_b = bcast[:, 5 * F:5 * F + N]

    ime = _dotbf(edge_ref[0], WimeT[...]) + bime[...]
    ime = ime * d_b

    iota = jax.lax.broadcasted_iota(jnp.int32, (TK, N), 1).astype(jnp.float32)
    onehot = jnp.where(idx_b == iota, 1.0, 0.0).astype(jnp.bfloat16)
    imn_f = _dotbf(onehot, imn_full)
    dr_f = _dotbf(onehot, drtab_ref[0])

    imn_i = jnp.broadcast_to(imn_tile[:, None, :], (T, K, F)).reshape(TK, F)
    msg = ime * imn_i * imn_f
    x_new = x_tile + (msg * mask_b).reshape(T, K, F).sum(axis=1)

    s_b = _dot(msg, Wemcb[...]) * mask_b
    s1 = s_b[:, 0:1]
    w3 = (s1 * cols[:, 1:2]) * cols[:, 2:5]
    eqF_new = eqF_ref[0] + w3.reshape(T, K, 3).sum(axis=1)

    emf = mlp2(msg, W1emf, b1emf, W2emf, b2emf)
    h = _dot(msg, W1eme[...])
    h = h * jax.nn.sigmoid(h)
    eme = _dot(h, W2eme[...]) * mask_b

    esc = mlp2(x_new, W1esc, b1esc, W2esc, b2esc)
    isc = mlp2(x_new, W1isc, b1isc, W2isc, b2isc)

    dot3 = jnp.zeros((T, F), jnp.float32)
    for c in range(3):
        w_c = s_b * dv_b[:, c * F:(c + 1) * F]
        eqf_c = eqf_ref[0, :, c, :] + (emf * w_c).reshape(T, K, F).sum(axis=1)
        dr_c = drtile_ref[0][:, c * F:(c + 1) * F]
        gdr_c = dr_f[:, c * F:(c + 1) * F]
        eqdr_c = dr_c + (eme * gdr_c).reshape(T, K, F).sum(axis=1)
        eqdr_c = eqdr_c + esc * eqf_c
        out_eqf[0, :, c, :] = eqf_c
        out_eqdr[0, :, c, :] = eqdr_c
        dot3 = dot3 + eqdr_c * eqf_c

    out_x[0] = x_new + isc * dot3
    out_eqF[0] = eqF_new


def kernel(invariant_node, equivariant_node_F, equivariant_node_f,
           equivariant_node_dr, invariant_edge, neighbor_mask, distances,
           distance_vectors, neighbor_indices, params):
    B, N, F = invariant_node.shape
    K = neighbor_indices.shape[-1]
    NB = invariant_edge.shape[-1]
    T = 64
    TK = T * K
    p = params

    edge_flat = invariant_edge.reshape(B, N * K, NB)
    cols = jnp.concatenate([
        (0.5 * (jnp.cos(jnp.pi * distances / _CUTOFF) + 1.0)
         * (distances < _CUTOFF)).reshape(B, N * K, 1),
        neighbor_mask.reshape(B, N * K, 1),
        distance_vectors.reshape(B, N * K, 3),
        neighbor_indices.astype(jnp.float32).reshape(B, N * K, 1),
    ], axis=-1)
    dr_tab = equivariant_node_dr.reshape(B, N, 3 * F)

    def b2(v):
        return v.reshape(1, F)

    weights = [
        p["W_ime"].T, p["b_ime"].reshape(1, F),
        p["imn_W1"].T, b2(p["imn_b1"]), p["imn_W2"].T, b2(p["imn_b2"]),
        jnp.broadcast_to(p["W_emc"].T, (F, F)),
        p["emf_W1"].T, b2(p["emf_b1"]), p["emf_W2"].T, b2(p["emf_b2"]),
        p["eme_W1"].T, p["eme_W2"].T,
        p["esc_W1"].T, b2(p["esc_b1"]), p["esc_W2"].T, b2(p["esc_b2"]),
        p["isc_W1"].T, b2(p["isc_b1"]), p["isc_W2"].T, b2(p["isc_b2"]),
    ]

    def wspec(w):
        shp = w.shape
        return pl.BlockSpec(shp, lambda b, t: (0,) * len(shp))

    grid = (B, N // T)
    in_specs = [
        pl.BlockSpec((1, N, F), lambda b, t: (b, 0, 0)),
        pl.BlockSpec((1, T, F), lambda b, t: (b, t, 0)),
        pl.BlockSpec((1, T, 3), lambda b, t: (b, t, 0)),
        pl.BlockSpec((1, T, 3, F), lambda b, t: (b, t, 0, 0)),
        pl.BlockSpec((1, N, 3 * F), lambda b, t: (b, 0, 0)),
        pl.BlockSpec((1, T, 3 * F), lambda b, t: (b, t, 0)),
        pl.BlockSpec((1, TK, NB), lambda b, t: (b, t, 0)),
        pl.BlockSpec((1, TK, 6), lambda b, t: (b, t, 0)),
    ] + [wspec(w) for w in weights]

    out_specs = [
        pl.BlockSpec((1, T, F), lambda b, t: (b, t, 0)),
        pl.BlockSpec((1, T, 3), lambda b, t: (b, t, 0)),
        pl.BlockSpec((1, T, 3, F), lambda b, t: (b, t, 0, 0)),
        pl.BlockSpec((1, T, 3, F), lambda b, t: (b, t, 0, 0)),
    ]
    out_shapes = [
        jax.ShapeDtypeStruct((B, N, F), jnp.float32),
        jax.ShapeDtypeStruct((B, N, 3), jnp.float32),
        jax.ShapeDtypeStruct((B, N, 3, F), jnp.float32),
        jax.ShapeDtypeStruct((B, N, 3, F), jnp.float32),
    ]

    import functools
    fn = functools.partial(_fused, T, K, F)
    outs = pl.pallas_call(
        fn,
        grid=grid,
        in_specs=in_specs,
        out_specs=out_specs,
        out_shape=out_shapes,
        compiler_params=pltpu.CompilerParams(
            dimension_semantics=("parallel", "parallel")),
    )(invariant_node, invariant_node, equivariant_node_F, equivariant_node_f,
      dr_tab, dr_tab, edge_flat, cols, *weights)

    return outs[0], outs[1], outs[2], outs[3]
